# baseline (speedup 1.0000x reference)
import functools

import jax
import jax.numpy as jnp
from jax import lax
from jax.experimental import pallas as pl
from jax.experimental.pallas import tpu as pltpu
from jax.experimental.pallas import tpu_sc as plsc


@functools.lru_cache(maxsize=None)
def _build_kernel(B: int, S: int, D: int):
    info = plsc.get_sparse_core_info()
    NC, NS, L = info.num_cores, info.num_subcores, info.num_lanes
    assert B % NC == 0 and B // NC <= NS
    assert S % L == 0
    bpc = B // NC

    mesh = plsc.VectorSubcoreMesh(core_axis_name="c", subcore_axis_name="s")

    @functools.partial(
        pl.kernel,
        mesh=mesh,
        out_type=jax.ShapeDtypeStruct((B, D), jnp.float32),
        scratch_types=[
            pltpu.VMEM((S,), jnp.float32),
            pltpu.VMEM((L, D), jnp.float32),
            pltpu.SemaphoreType.DMA,
        ],
    )
    def last_token_pool(enc_hbm, masks_hbm, out_hbm, mask_v, rows_v, sem):
        c = lax.axis_index("c")
        s = lax.axis_index("s")

        @pl.when(s < bpc)
        def _leader():
            b = c * bpc + s
            pltpu.sync_copy(masks_hbm.at[b], mask_v)

            def body(j, acc):
                v = mask_v[pl.ds(j * L, L)]
                return acc + v.astype(jnp.int32)

            acc = lax.fori_loop(
                0, S // L, body, jnp.zeros((L,), jnp.int32), unroll=8
            )
            gather_dnums = lax.GatherDimensionNumbers(
                offset_dims=(), collapsed_slice_dims=(0,), start_index_map=(0,)
            )
            for shift in (8, 4, 2, 1):
                perm = (lax.iota(jnp.int32, L) + shift) % L
                rot = lax.gather(
                    acc,
                    perm[:, None],
                    dimension_numbers=gather_dnums,
                    slice_sizes=(1,),
                    mode=lax.GatherScatterMode.PROMISE_IN_BOUNDS,
                )
                acc = acc + rot
            idx_vec = jnp.clip(acc - 1, 0, S - 1) + b * S
            pltpu.async_copy(enc_hbm.at[idx_vec], rows_v, sem).wait()
            pltpu.sync_copy(rows_v.at[pl.ds(0, 1)], out_hbm.at[pl.ds(b, 1)])

    return last_token_pool


@jax.jit
def kernel(encoded_inputs, input_masks):
    B, S, D = encoded_inputs.shape
    enc_flat = encoded_inputs.reshape(B * S, D)
    return _build_kernel(B, S, D)(enc_flat, input_masks)


# speedup vs baseline: 1.2106x; 1.2106x over previous
import functools

import jax
import jax.numpy as jnp
from jax import lax
from jax.experimental import pallas as pl
from jax.experimental.pallas import tpu as pltpu
from jax.experimental.pallas import tpu_sc as plsc


@functools.lru_cache(maxsize=None)
def _build_kernel(B: int, S: int, D: int):
    info = plsc.get_sparse_core_info()
    NC, NS, L = info.num_cores, info.num_subcores, info.num_lanes
    NW = NC * NS
    assert NW % B == 0
    SEG = NW // B
    assert SEG <= NS and NS % SEG == 0
    C = S // SEG
    assert C % L == 0

    mesh = plsc.VectorSubcoreMesh(core_axis_name="c", subcore_axis_name="s")

    @functools.partial(
        pl.kernel,
        mesh=mesh,
        out_type=jax.ShapeDtypeStruct((B, D), jnp.float32),
        scratch_types=[
            pltpu.VMEM((C,), jnp.float32),
            pltpu.SMEM((1,), jnp.int32),
        ],
    )
    def last_token_pool(enc_hbm, masks_hbm, out_hbm, mask_v, tot_smem):
        c = lax.axis_index("c")
        s = lax.axis_index("s")
        w = c * NS + s
        b = w // SEG
        seg = w % SEG
        leader = (s // SEG) * SEG

        tot_smem[0] = 0
        plsc.subcore_barrier()

        pltpu.sync_copy(masks_hbm.at[b, pl.ds(seg * C, C)], mask_v)
        acc = jnp.zeros((L,), jnp.int32)
        for j in range(C // L):
            acc = acc + mask_v[pl.ds(j * L, L)].astype(jnp.int32)
        gather_dnums = lax.GatherDimensionNumbers(
            offset_dims=(), collapsed_slice_dims=(0,), start_index_map=(0,)
        )
        shift = L // 2
        while shift >= 1:
            perm = (lax.iota(jnp.int32, L) + shift) % L
            acc = acc + lax.gather(
                acc,
                perm[:, None],
                dimension_numbers=gather_dnums,
                slice_sizes=(1,),
                mode=lax.GatherScatterMode.PROMISE_IN_BOUNDS,
            )
            shift //= 2
        plsc.fetch_and_add(tot_smem, acc[0], subcore_id=leader)
        plsc.subcore_barrier()

        @pl.when(seg == 0)
        def _leader():
            total = tot_smem[0]
            idx = jnp.clip(total - 1, 0, S - 1)
            pltpu.sync_copy(
                enc_hbm.at[pl.ds(b * S + idx, 1)], out_hbm.at[pl.ds(b, 1)]
            )

    return last_token_pool


@jax.jit
def kernel(encoded_inputs, input_masks):
    B, S, D = encoded_inputs.shape
    enc_flat = encoded_inputs.reshape(B * S, D)
    return _build_kernel(B, S, D)(enc_flat, input_masks)


# speedup vs baseline: 1.2568x; 1.0382x over previous
import functools

import jax
import jax.numpy as jnp
from jax import lax
from jax.experimental import pallas as pl
from jax.experimental.pallas import tpu as pltpu
from jax.experimental.pallas import tpu_sc as plsc


@functools.lru_cache(maxsize=None)
def _build_kernel(B: int, S: int, D: int):
    info = plsc.get_sparse_core_info()
    NC, NS, L = 1, info.num_subcores, info.num_lanes
    NW = NC * NS
    assert NW % B == 0
    SEG = NW // B
    assert SEG <= NS and NS % SEG == 0
    C = S // SEG
    assert C % L == 0

    mesh = plsc.VectorSubcoreMesh(
        core_axis_name="c", subcore_axis_name="s", num_cores=NC
    )

    @functools.partial(
        pl.kernel,
        mesh=mesh,
        out_type=jax.ShapeDtypeStruct((B, D), jnp.float32),
        scratch_types=[
            pltpu.VMEM((C,), jnp.float32),
            pltpu.SMEM((1,), jnp.int32),
        ],
    )
    def last_token_pool(enc_hbm, masks_hbm, out_hbm, mask_v, tot_smem):
        c = lax.axis_index("c")
        s = lax.axis_index("s")
        w = c * NS + s
        b = w // SEG
        seg = w % SEG
        leader = (s // SEG) * SEG

        tot_smem[0] = 0
        plsc.subcore_barrier()

        pltpu.sync_copy(masks_hbm.at[b, pl.ds(seg * C, C)], mask_v)
        acc = jnp.zeros((L,), jnp.int32)
        for j in range(C // L):
            acc = acc + mask_v[pl.ds(j * L, L)].astype(jnp.int32)
        gather_dnums = lax.GatherDimensionNumbers(
            offset_dims=(), collapsed_slice_dims=(0,), start_index_map=(0,)
        )
        shift = L // 2
        while shift >= 1:
            perm = (lax.iota(jnp.int32, L) + shift) % L
            acc = acc + lax.gather(
                acc,
                perm[:, None],
                dimension_numbers=gather_dnums,
                slice_sizes=(1,),
                mode=lax.GatherScatterMode.PROMISE_IN_BOUNDS,
            )
            shift //= 2
        plsc.fetch_and_add(tot_smem, acc[0], subcore_id=leader)
        plsc.subcore_barrier()

        @pl.when(seg == 0)
        def _leader():
            total = tot_smem[0]
            idx = jnp.clip(total - 1, 0, S - 1)
            pltpu.sync_copy(
                enc_hbm.at[pl.ds(b * S + idx, 1)], out_hbm.at[pl.ds(b, 1)]
            )

    return last_token_pool


@jax.jit
def kernel(encoded_inputs, input_masks):
    B, S, D = encoded_inputs.shape
    enc_flat = encoded_inputs.reshape(B * S, D)
    return _build_kernel(B, S, D)(enc_flat, input_masks)


# speedup vs baseline: 9.9557x; 7.9214x over previous
import functools

import jax
import jax.numpy as jnp
from jax.experimental import pallas as pl
from jax.experimental.pallas import tpu as pltpu


def _pool_body(S, B, mask_ref, enc_hbm, out_ref, sem):
    copies = []
    for b in range(B):
        total = jnp.sum(mask_ref[b, :].astype(jnp.int32))
        idx = jnp.clip(total - 1, 0, S - 1)
        copies.append(
            pltpu.make_async_copy(
                enc_hbm.at[b, pl.ds(idx, 1)],
                out_ref.at[pl.ds(b, 1)],
                sem.at[b],
            )
        )
        copies[-1].start()
    for c in copies:
        c.wait()


@functools.lru_cache(maxsize=None)
def _build_kernel(B: int, S: int, D: int):
    return pl.pallas_call(
        functools.partial(_pool_body, S, B),
        grid=(),
        in_specs=[
            pl.BlockSpec(memory_space=pltpu.VMEM),
            pl.BlockSpec(memory_space=pl.ANY),
        ],
        out_specs=pl.BlockSpec(memory_space=pltpu.VMEM),
        out_shape=jax.ShapeDtypeStruct((B, D), jnp.float32),
        scratch_shapes=[pltpu.SemaphoreType.DMA((B,))],
    )


@jax.jit
def kernel(encoded_inputs, input_masks):
    B, S, D = encoded_inputs.shape
    return _build_kernel(B, S, D)(input_masks, encoded_inputs)
